# Initial kernel scaffold; baseline (speedup 1.0000x reference)
import jax
import jax.numpy as jnp
from jax import lax
from jax.experimental import pallas as pl
from jax.experimental.pallas import tpu as pltpu
from jax.experimental.pallas import tpu_sc as plsc

_B = 16384
_NH = 512
_ED = 16

_NC = 2
_NS = 16
_NW = _NC * _NS
_BPW = _B // _NW


def _sc_gather_body(temb_hbm, memb_hbm, tid_hbm, mid_hbm,
                    tout_hbm, mout_hbm,
                    tid_v, mid_v, trows_v, mrows_v, sem_t, sem_m):
    wid = lax.axis_index("s") * _NC + lax.axis_index("c")
    base = wid * _BPW
    pltpu.sync_copy(tid_hbm.at[pl.ds(base, _BPW)], tid_v)
    pltpu.sync_copy(mid_hbm.at[pl.ds(base, _BPW)], mid_v)
    c_t = pltpu.async_copy(temb_hbm.at[tid_v], trows_v, sem_t)
    c_m = pltpu.async_copy(memb_hbm.at[mid_v], mrows_v, sem_m)
    c_t.wait()
    c_m.wait()
    pltpu.sync_copy(trows_v, tout_hbm.at[pl.ds(base, _BPW)])
    pltpu.sync_copy(mrows_v, mout_hbm.at[pl.ds(base, _BPW)])


_sc_gather = pl.kernel(
    _sc_gather_body,
    out_type=[
        jax.ShapeDtypeStruct((_B, _ED), jnp.float32),
        jax.ShapeDtypeStruct((_B, _ED), jnp.float32),
    ],
    mesh=plsc.VectorSubcoreMesh(
        core_axis_name="c", subcore_axis_name="s",
        num_cores=_NC, num_subcores=_NS),
    scratch_types=[
        pltpu.VMEM((_BPW,), jnp.int32),
        pltpu.VMEM((_BPW,), jnp.int32),
        pltpu.VMEM((_BPW, _ED), jnp.float32),
        pltpu.VMEM((_BPW, _ED), jnp.float32),
        pltpu.SemaphoreType.DMA,
        pltpu.SemaphoreType.DMA,
    ],
)

_BLK = 2048


def _tc_head_body(h_ref, t_ref, m_ref, wh_ref, wt_ref, wm_ref, b_ref, o_ref):
    acc = jnp.dot(h_ref[...], wh_ref[...], preferred_element_type=jnp.float32)
    acc = acc + jnp.dot(t_ref[...], wt_ref[...],
                        preferred_element_type=jnp.float32)
    acc = acc + jnp.dot(m_ref[...], wm_ref[...],
                        preferred_element_type=jnp.float32)
    o_ref[...] = acc + b_ref[0]


_tc_head = pl.pallas_call(
    _tc_head_body,
    grid=(_B // _BLK,),
    in_specs=[
        pl.BlockSpec((_BLK, _NH), lambda i: (i, 0)),
        pl.BlockSpec((_BLK, _ED), lambda i: (i, 0)),
        pl.BlockSpec((_BLK, _ED), lambda i: (i, 0)),
        pl.BlockSpec((_NH, 1), lambda i: (0, 0)),
        pl.BlockSpec((_ED, 1), lambda i: (0, 0)),
        pl.BlockSpec((_ED, 1), lambda i: (0, 0)),
        pl.BlockSpec(memory_space=pltpu.SMEM),
    ],
    out_specs=pl.BlockSpec((_BLK, 1), lambda i: (i, 0)),
    out_shape=jax.ShapeDtypeStruct((_B, 1), jnp.float32),
)


@jax.jit
def kernel(h, teacher_id, materia_id, teacher_emb, materia_emb, W, b):
    tid = teacher_id.astype(jnp.int32)
    mid = materia_id.astype(jnp.int32)
    t_rows, m_rows = _sc_gather(teacher_emb, materia_emb, tid, mid)
    wh = W[0, :_NH].reshape(_NH, 1)
    wt = W[0, _NH:_NH + _ED].reshape(_ED, 1)
    wm = W[0, _NH + _ED:].reshape(_ED, 1)
    out = _tc_head(h, t_rows, m_rows, wh, wt, wm, b)
    return out.reshape(_B)


# speedup vs baseline: 1.5511x; 1.5511x over previous
import jax
import jax.numpy as jnp
from jax import lax
from jax.experimental import pallas as pl
from jax.experimental.pallas import tpu as pltpu
from jax.experimental.pallas import tpu_sc as plsc

_B = 16384
_NH = 512
_ED = 16

_NC = 2
_NS = 16
_NW = _NC * _NS
_BPW = _B // _NW


def _sc_gather_body(temb_hbm, memb_hbm, tid_hbm, mid_hbm,
                    tout_hbm, mout_hbm,
                    tid_v, mid_v, trows_v, mrows_v, sem_t, sem_m):
    wid = lax.axis_index("s") * _NC + lax.axis_index("c")
    base = wid * _BPW
    pltpu.sync_copy(tid_hbm.at[pl.ds(base, _BPW)], tid_v)
    pltpu.sync_copy(mid_hbm.at[pl.ds(base, _BPW)], mid_v)
    c_t = pltpu.async_copy(temb_hbm.at[tid_v], trows_v, sem_t)
    c_m = pltpu.async_copy(memb_hbm.at[mid_v], mrows_v, sem_m)
    c_t.wait()
    c_m.wait()
    pltpu.sync_copy(trows_v, tout_hbm.at[pl.ds(base, _BPW)])
    pltpu.sync_copy(mrows_v, mout_hbm.at[pl.ds(base, _BPW)])


_SC_GATHER = None


def _get_sc_gather():
    global _SC_GATHER
    if _SC_GATHER is None:
        _SC_GATHER = pl.kernel(
            _sc_gather_body,
            out_type=[
                jax.ShapeDtypeStruct((_B, _ED), jnp.float32),
                jax.ShapeDtypeStruct((_B, _ED), jnp.float32),
            ],
            mesh=plsc.VectorSubcoreMesh(
                core_axis_name="c", subcore_axis_name="s",
                num_cores=_NC, num_subcores=_NS),
            scratch_types=[
                pltpu.VMEM((_BPW,), jnp.int32),
                pltpu.VMEM((_BPW,), jnp.int32),
                pltpu.VMEM((_BPW, _ED), jnp.float32),
                pltpu.VMEM((_BPW, _ED), jnp.float32),
                pltpu.SemaphoreType.DMA,
                pltpu.SemaphoreType.DMA,
            ],
            compiler_params=pltpu.CompilerParams(use_tc_tiling_on_sc=False),
        )
    return _SC_GATHER

_BLK = 2048


def _tc_head_body(h_ref, t_ref, m_ref, wh_ref, wt_ref, wm_ref, b_ref, o_ref):
    acc = jnp.dot(h_ref[...], wh_ref[...], preferred_element_type=jnp.float32)
    acc = acc + jnp.dot(t_ref[...], wt_ref[...],
                        preferred_element_type=jnp.float32)
    acc = acc + jnp.dot(m_ref[...], wm_ref[...],
                        preferred_element_type=jnp.float32)
    o_ref[...] = acc + b_ref[0]


_tc_head = pl.pallas_call(
    _tc_head_body,
    grid=(_B // _BLK,),
    in_specs=[
        pl.BlockSpec((_BLK, _NH), lambda i: (i, 0)),
        pl.BlockSpec((_BLK, _ED), lambda i: (i, 0)),
        pl.BlockSpec((_BLK, _ED), lambda i: (i, 0)),
        pl.BlockSpec((_NH, 1), lambda i: (0, 0)),
        pl.BlockSpec((_ED, 1), lambda i: (0, 0)),
        pl.BlockSpec((_ED, 1), lambda i: (0, 0)),
        pl.BlockSpec(memory_space=pltpu.SMEM),
    ],
    out_specs=pl.BlockSpec((_BLK, 1), lambda i: (i, 0)),
    out_shape=jax.ShapeDtypeStruct((_B, 1), jnp.float32),
)


@jax.jit
def kernel(h, teacher_id, materia_id, teacher_emb, materia_emb, W, b):
    tid = teacher_id.astype(jnp.int32)
    mid = materia_id.astype(jnp.int32)
    t_rows, m_rows = _get_sc_gather()(teacher_emb, materia_emb, tid, mid)
    wh = W[0, :_NH].reshape(_NH, 1)
    wt = W[0, _NH:_NH + _ED].reshape(_ED, 1)
    wm = W[0, _NH + _ED:].reshape(_ED, 1)
    out = _tc_head(h, t_rows, m_rows, wh, wt, wm, b)
    return out.reshape(_B)


# speedup vs baseline: 1.8226x; 1.1750x over previous
import jax
import jax.numpy as jnp
from jax import lax
from jax.experimental import pallas as pl
from jax.experimental.pallas import tpu as pltpu
from jax.experimental.pallas import tpu_sc as plsc

_B = 16384
_NH = 512
_ED = 16

_NC = 2
_NS = 16
_NW = _NC * _NS
_BPW = _B // _NW
_NG = _BPW // 16


def _sc_gather_body(temb_hbm, memb_hbm, tid_hbm, mid_hbm, wtm_hbm,
                    g_hbm,
                    tid_v, mid_v, trows_v, mrows_v, wtm_v, g_v,
                    sem_t, sem_m):
    wid = lax.axis_index("s") * _NC + lax.axis_index("c")
    base = wid * _BPW
    pltpu.sync_copy(tid_hbm.at[pl.ds(base, _BPW)], tid_v)
    pltpu.sync_copy(mid_hbm.at[pl.ds(base, _BPW)], mid_v)
    pltpu.sync_copy(wtm_hbm, wtm_v)
    c_t = pltpu.async_copy(temb_hbm.at[tid_v], trows_v, sem_t)
    c_m = pltpu.async_copy(memb_hbm.at[mid_v], mrows_v, sem_m)
    c_t.wait()
    c_m.wait()

    wt_vec = wtm_v[pl.ds(0, 16)]
    wm_vec = wtm_v[pl.ds(16, 16)]

    def group(i, _):
        rows = jnp.full((16,), i * 16, jnp.int32) + lax.iota(jnp.int32, 16)
        acc = jnp.zeros((16,), jnp.float32)
        for j in range(_ED):
            col = jnp.full((16,), j, jnp.int32)
            acc = acc + plsc.load_gather(trows_v, [rows, col]) * wt_vec[j]
            acc = acc + plsc.load_gather(mrows_v, [rows, col]) * wm_vec[j]
        g_v[pl.ds(i * 16, 16)] = acc
        return ()

    lax.fori_loop(0, _NG, group, (), unroll=2)
    pltpu.sync_copy(g_v, g_hbm.at[pl.ds(base, _BPW)])


_SC_GATHER = None


def _get_sc_gather():
    global _SC_GATHER
    if _SC_GATHER is None:
        _SC_GATHER = pl.kernel(
            _sc_gather_body,
            out_type=jax.ShapeDtypeStruct((_B,), jnp.float32),
            mesh=plsc.VectorSubcoreMesh(
                core_axis_name="c", subcore_axis_name="s",
                num_cores=_NC, num_subcores=_NS),
            scratch_types=[
                pltpu.VMEM((_BPW,), jnp.int32),
                pltpu.VMEM((_BPW,), jnp.int32),
                pltpu.VMEM((_BPW, _ED), jnp.float32),
                pltpu.VMEM((_BPW, _ED), jnp.float32),
                pltpu.VMEM((2 * _ED,), jnp.float32),
                pltpu.VMEM((_BPW,), jnp.float32),
                pltpu.SemaphoreType.DMA,
                pltpu.SemaphoreType.DMA,
            ],
            compiler_params=pltpu.CompilerParams(
                use_tc_tiling_on_sc=False, needs_layout_passes=False),
        )
    return _SC_GATHER


_BLK = 2048


def _tc_head_body(h_ref, wh_ref, g_ref, b_ref, o_ref):
    acc = jnp.dot(h_ref[...], wh_ref[...], preferred_element_type=jnp.float32)
    o_ref[...] = acc[:, 0] + g_ref[...] + b_ref[0]


_tc_head = pl.pallas_call(
    _tc_head_body,
    grid=(_B // _BLK,),
    in_specs=[
        pl.BlockSpec((_BLK, _NH), lambda i: (i, 0)),
        pl.BlockSpec((_NH, 1), lambda i: (0, 0)),
        pl.BlockSpec((_BLK,), lambda i: (i,)),
        pl.BlockSpec(memory_space=pltpu.SMEM),
    ],
    out_specs=pl.BlockSpec((_BLK,), lambda i: (i,)),
    out_shape=jax.ShapeDtypeStruct((_B,), jnp.float32),
)


@jax.jit
def kernel(h, teacher_id, materia_id, teacher_emb, materia_emb, W, b):
    tid = teacher_id.astype(jnp.int32)
    mid = materia_id.astype(jnp.int32)
    wtm = W[0, _NH:]
    g = _get_sc_gather()(teacher_emb, materia_emb, tid, mid, wtm)
    wh = W[0, :_NH].reshape(_NH, 1)
    return _tc_head(h, wh, g, b)


# speedup vs baseline: 1.9524x; 1.0713x over previous
import jax
import jax.numpy as jnp
from jax import lax
from jax.experimental import pallas as pl
from jax.experimental.pallas import tpu as pltpu
from jax.experimental.pallas import tpu_sc as plsc

_B = 16384
_NH = 512
_ED = 16

_NC = 2
_NS = 16
_NW = _NC * _NS
_BPW = _B // _NW
_NG = _BPW // 16


def _sc_gather_body(temb_hbm, memb_hbm, tid_hbm, mid_hbm, wtm_hbm,
                    g_hbm,
                    tid_v, mid_v, trows_v, mrows_v, wtm_v, g_v,
                    sem_t, sem_m):
    wid = lax.axis_index("s") * _NC + lax.axis_index("c")
    base = wid * _BPW
    pltpu.sync_copy(tid_hbm.at[pl.ds(base, _BPW)], tid_v)
    pltpu.sync_copy(mid_hbm.at[pl.ds(base, _BPW)], mid_v)
    pltpu.sync_copy(wtm_hbm, wtm_v)
    c_t = pltpu.async_copy(temb_hbm.at[tid_v], trows_v, sem_t)
    c_m = pltpu.async_copy(memb_hbm.at[mid_v], mrows_v, sem_m)
    c_t.wait()
    c_m.wait()

    wt_vec = wtm_v[pl.ds(0, 16)]
    wm_vec = wtm_v[pl.ds(16, 16)]

    def group(i, _):
        rows = jnp.full((16,), i * 16, jnp.int32) + lax.iota(jnp.int32, 16)
        acc = jnp.zeros((16,), jnp.float32)
        for j in range(_ED):
            col = jnp.full((16,), j, jnp.int32)
            acc = acc + plsc.load_gather(trows_v, [rows, col]) * wt_vec[j]
            acc = acc + plsc.load_gather(mrows_v, [rows, col]) * wm_vec[j]
        g_v[pl.ds(i * 16, 16)] = acc
        return ()

    lax.fori_loop(0, _NG, group, (), unroll=2)
    pltpu.sync_copy(g_v, g_hbm.at[pl.ds(base, _BPW)])


_SC_GATHER = None


def _get_sc_gather():
    global _SC_GATHER
    if _SC_GATHER is None:
        _SC_GATHER = pl.kernel(
            _sc_gather_body,
            out_type=jax.ShapeDtypeStruct((_B,), jnp.float32),
            mesh=plsc.VectorSubcoreMesh(
                core_axis_name="c", subcore_axis_name="s",
                num_cores=_NC, num_subcores=_NS),
            scratch_types=[
                pltpu.VMEM((_BPW,), jnp.int32),
                pltpu.VMEM((_BPW,), jnp.int32),
                pltpu.VMEM((_BPW, _ED), jnp.float32),
                pltpu.VMEM((_BPW, _ED), jnp.float32),
                pltpu.VMEM((2 * _ED,), jnp.float32),
                pltpu.VMEM((_BPW,), jnp.float32),
                pltpu.SemaphoreType.DMA,
                pltpu.SemaphoreType.DMA,
            ],
            compiler_params=pltpu.CompilerParams(
                use_tc_tiling_on_sc=False, needs_layout_passes=False),
        )
    return _SC_GATHER


_BLK = 2048


def _tc_head_body(h_ref, wh_ref, b_ref, o_ref):
    acc = jnp.dot(h_ref[...], wh_ref[...], preferred_element_type=jnp.float32)
    o_ref[...] = acc[:, 0] + b_ref[0]


_tc_head = pl.pallas_call(
    _tc_head_body,
    grid=(_B // _BLK,),
    in_specs=[
        pl.BlockSpec((_BLK, _NH), lambda i: (i, 0)),
        pl.BlockSpec((_NH, 1), lambda i: (0, 0)),
        pl.BlockSpec(memory_space=pltpu.SMEM),
    ],
    out_specs=pl.BlockSpec((_BLK,), lambda i: (i,)),
    out_shape=jax.ShapeDtypeStruct((_B,), jnp.float32),
)


@jax.jit
def kernel(h, teacher_id, materia_id, teacher_emb, materia_emb, W, b):
    tid = teacher_id.astype(jnp.int32)
    mid = materia_id.astype(jnp.int32)
    wtm = W[0, _NH:]
    g = _get_sc_gather()(teacher_emb, materia_emb, tid, mid, wtm)
    wh = W[0, :_NH].reshape(_NH, 1)
    oh = _tc_head(h, wh, b)
    return oh + g


# speedup vs baseline: 2.9693x; 1.5208x over previous
import jax
import jax.numpy as jnp
from jax import lax
from jax.experimental import pallas as pl
from jax.experimental.pallas import tpu as pltpu
from jax.experimental.pallas import tpu_sc as plsc

_B = 16384
_NH = 512
_ED = 16
_NT = 100000
_NM = 1000

_NC = 2
_NS = 16
_NW = _NC * _NS
_BPW = _B // _NW


def _sc_gather_body(ts_hbm, ms_hbm, tid_hbm, mid_hbm,
                    g_hbm,
                    tid_v, mid_v, ts_v, ms_v, sem_t, sem_m):
    wid = lax.axis_index("s") * _NC + lax.axis_index("c")
    base = wid * _BPW
    pltpu.sync_copy(tid_hbm.at[pl.ds(base, _BPW)], tid_v)
    pltpu.sync_copy(mid_hbm.at[pl.ds(base, _BPW)], mid_v)
    c_t = pltpu.async_copy(ts_hbm.at[tid_v], ts_v, sem_t)
    c_m = pltpu.async_copy(ms_hbm.at[mid_v], ms_v, sem_m)
    c_t.wait()
    c_m.wait()
    for i in range(_BPW // 16):
        sl = pl.ds(i * 16, 16)
        ts_v[sl] = ts_v[sl] + ms_v[sl]
    pltpu.sync_copy(ts_v, g_hbm.at[pl.ds(base, _BPW)])


_SC_GATHER = None


def _get_sc_gather():
    global _SC_GATHER
    if _SC_GATHER is None:
        _SC_GATHER = pl.kernel(
            _sc_gather_body,
            out_type=jax.ShapeDtypeStruct((_B,), jnp.float32),
            mesh=plsc.VectorSubcoreMesh(
                core_axis_name="c", subcore_axis_name="s",
                num_cores=_NC, num_subcores=_NS),
            scratch_types=[
                pltpu.VMEM((_BPW,), jnp.int32),
                pltpu.VMEM((_BPW,), jnp.int32),
                pltpu.VMEM((_BPW,), jnp.float32),
                pltpu.VMEM((_BPW,), jnp.float32),
                pltpu.SemaphoreType.DMA,
                pltpu.SemaphoreType.DMA,
            ],
            compiler_params=pltpu.CompilerParams(
                use_tc_tiling_on_sc=False, needs_layout_passes=False),
        )
    return _SC_GATHER


_TBLK = 16384


def _tc_scores_body(tt_ref, mt_ref, wt_ref, wm_ref, ts_ref, ms_ref):
    ts_ref[...] = jnp.sum(tt_ref[...] * wt_ref[...], axis=0)

    @pl.when(pl.program_id(0) == 0)
    def _():
        ms_ref[...] = jnp.sum(mt_ref[...] * wm_ref[...], axis=0)


_tc_scores = pl.pallas_call(
    _tc_scores_body,
    grid=(pl.cdiv(_NT, _TBLK),),
    in_specs=[
        pl.BlockSpec((_ED, _TBLK), lambda i: (0, i)),
        pl.BlockSpec((_ED, _NM), lambda i: (0, 0)),
        pl.BlockSpec((_ED, 1), lambda i: (0, 0)),
        pl.BlockSpec((_ED, 1), lambda i: (0, 0)),
    ],
    out_specs=[
        pl.BlockSpec((_TBLK,), lambda i: (i,)),
        pl.BlockSpec((_NM,), lambda i: (0,)),
    ],
    out_shape=[
        jax.ShapeDtypeStruct((_NT,), jnp.float32),
        jax.ShapeDtypeStruct((_NM,), jnp.float32),
    ],
)

_BLK = 2048


def _tc_head_body(h_ref, wh_ref, b_ref, o_ref):
    acc = jnp.dot(h_ref[...], wh_ref[...], preferred_element_type=jnp.float32)
    o_ref[...] = acc[:, 0] + b_ref[0]


_tc_head = pl.pallas_call(
    _tc_head_body,
    grid=(_B // _BLK,),
    in_specs=[
        pl.BlockSpec((_BLK, _NH), lambda i: (i, 0)),
        pl.BlockSpec((_NH, 1), lambda i: (0, 0)),
        pl.BlockSpec(memory_space=pltpu.SMEM),
    ],
    out_specs=pl.BlockSpec((_BLK,), lambda i: (i,)),
    out_shape=jax.ShapeDtypeStruct((_B,), jnp.float32),
)


@jax.jit
def kernel(h, teacher_id, materia_id, teacher_emb, materia_emb, W, b):
    tid = teacher_id.astype(jnp.int32)
    mid = materia_id.astype(jnp.int32)
    wt2 = W[0, _NH:_NH + _ED].reshape(_ED, 1)
    wm2 = W[0, _NH + _ED:].reshape(_ED, 1)
    ts, ms = _tc_scores(teacher_emb.T, materia_emb.T, wt2, wm2)
    g = _get_sc_gather()(ts, ms, tid, mid)
    wh = W[0, :_NH].reshape(_NH, 1)
    oh = _tc_head(h, wh, b)
    return oh + g


# speedup vs baseline: 3.5248x; 1.1871x over previous
import jax
import jax.numpy as jnp
from jax import lax
from jax.experimental import pallas as pl
from jax.experimental.pallas import tpu as pltpu
from jax.experimental.pallas import tpu_sc as plsc

_B = 16384
_NH = 512
_ED = 16
_NT = 100000
_NM = 1000

_NC = 2
_NS = 16
_NW = _NC * _NS
_BPW = _B // _NW


def _sc_gather_body(ts_hbm, ms_hbm, tid_hbm, mid_hbm,
                    g_hbm,
                    tid_v, mid_v, ts_v, ms_tab_v, g_v, stage_v, ts_spm,
                    sem_t):
    sid = lax.axis_index("s")
    wid = sid * _NC + lax.axis_index("c")
    base = wid * _BPW
    pltpu.sync_copy(tid_hbm.at[pl.ds(base, _BPW)], tid_v)
    pltpu.sync_copy(mid_hbm.at[pl.ds(base, _BPW)], mid_v)
    pltpu.sync_copy(ms_hbm, ms_tab_v)

    @pl.when(sid == 0)
    def _():
        pltpu.sync_copy(ts_hbm, stage_v)
        pltpu.sync_copy(stage_v, ts_spm)

    plsc.subcore_barrier()
    pltpu.async_copy(ts_spm.at[tid_v], ts_v, sem_t).wait()

    for i in range(_BPW // 16):
        sl = pl.ds(i * 16, 16)
        mvals = plsc.load_gather(ms_tab_v, [mid_v[sl]])
        g_v[sl] = ts_v[sl] + mvals
    pltpu.sync_copy(g_v, g_hbm.at[pl.ds(base, _BPW)])


_SC_GATHER = None


def _get_sc_gather():
    global _SC_GATHER
    if _SC_GATHER is None:
        _SC_GATHER = pl.kernel(
            _sc_gather_body,
            out_type=jax.ShapeDtypeStruct((_B,), jnp.float32),
            mesh=plsc.VectorSubcoreMesh(
                core_axis_name="c", subcore_axis_name="s",
                num_cores=_NC, num_subcores=_NS),
            scratch_types=[
                pltpu.VMEM((_BPW,), jnp.int32),
                pltpu.VMEM((_BPW,), jnp.int32),
                pltpu.VMEM((_BPW,), jnp.float32),
                pltpu.VMEM((_NM,), jnp.float32),
                pltpu.VMEM((_BPW,), jnp.float32),
                pltpu.VMEM((_NT,), jnp.float32),
                pltpu.VMEM_SHARED((_NT,), jnp.float32),
                pltpu.SemaphoreType.DMA,
            ],
            compiler_params=pltpu.CompilerParams(
                use_tc_tiling_on_sc=False, needs_layout_passes=False),
        )
    return _SC_GATHER


_TBLK = 16384


def _tc_scores_body(tt_ref, mt_ref, wt_ref, wm_ref, ts_ref, ms_ref):
    ts_ref[...] = jnp.sum(tt_ref[...] * wt_ref[...], axis=0)

    @pl.when(pl.program_id(0) == 0)
    def _():
        ms_ref[...] = jnp.sum(mt_ref[...] * wm_ref[...], axis=0)


_tc_scores = pl.pallas_call(
    _tc_scores_body,
    grid=(pl.cdiv(_NT, _TBLK),),
    in_specs=[
        pl.BlockSpec((_ED, _TBLK), lambda i: (0, i)),
        pl.BlockSpec((_ED, _NM), lambda i: (0, 0)),
        pl.BlockSpec((_ED, 1), lambda i: (0, 0)),
        pl.BlockSpec((_ED, 1), lambda i: (0, 0)),
    ],
    out_specs=[
        pl.BlockSpec((_TBLK,), lambda i: (i,)),
        pl.BlockSpec((_NM,), lambda i: (0,)),
    ],
    out_shape=[
        jax.ShapeDtypeStruct((_NT,), jnp.float32),
        jax.ShapeDtypeStruct((_NM,), jnp.float32),
    ],
)

_BLK = 2048


def _tc_head_body(h_ref, wh_ref, b_ref, o_ref):
    acc = jnp.dot(h_ref[...], wh_ref[...], preferred_element_type=jnp.float32)
    o_ref[...] = acc[:, 0] + b_ref[0]


_tc_head = pl.pallas_call(
    _tc_head_body,
    grid=(_B // _BLK,),
    in_specs=[
        pl.BlockSpec((_BLK, _NH), lambda i: (i, 0)),
        pl.BlockSpec((_NH, 1), lambda i: (0, 0)),
        pl.BlockSpec(memory_space=pltpu.SMEM),
    ],
    out_specs=pl.BlockSpec((_BLK,), lambda i: (i,)),
    out_shape=jax.ShapeDtypeStruct((_B,), jnp.float32),
)


@jax.jit
def kernel(h, teacher_id, materia_id, teacher_emb, materia_emb, W, b):
    tid = teacher_id.astype(jnp.int32)
    mid = materia_id.astype(jnp.int32)
    wt2 = W[0, _NH:_NH + _ED].reshape(_ED, 1)
    wm2 = W[0, _NH + _ED:].reshape(_ED, 1)
    ts, ms = _tc_scores(teacher_emb.T, materia_emb.T, wt2, wm2)
    g = _get_sc_gather()(ts, ms, tid, mid)
    wh = W[0, :_NH].reshape(_NH, 1)
    oh = _tc_head(h, wh, b)
    return oh + g


# speedup vs baseline: 3.9203x; 1.1122x over previous
import jax
import jax.numpy as jnp
from jax import lax
from jax.experimental import pallas as pl
from jax.experimental.pallas import tpu as pltpu
from jax.experimental.pallas import tpu_sc as plsc

_B = 16384
_NH = 512
_ED = 16
_NT = 100000
_NM = 1000

_NC = 2
_NS = 16
_NW = _NC * _NS
_BPW = _B // _NW


def _sc_gather_body(ts_hbm, ms_hbm, tid_hbm, mid_hbm,
                    g_hbm,
                    tid_v, mid_v, ts_v, ms_tab_v, g_v, stage_v, ts_spm,
                    sem_t):
    sid = lax.axis_index("s")
    wid = sid * _NC + lax.axis_index("c")
    base = wid * _BPW
    pltpu.sync_copy(tid_hbm.at[pl.ds(base, _BPW)], tid_v)
    pltpu.sync_copy(mid_hbm.at[pl.ds(base, _BPW)], mid_v)
    pltpu.sync_copy(ms_hbm, ms_tab_v)

    @pl.when(sid == 0)
    def _():
        pltpu.sync_copy(ts_hbm, stage_v)
        pltpu.sync_copy(stage_v, ts_spm)

    plsc.subcore_barrier()
    pltpu.async_copy(ts_spm.at[tid_v], ts_v, sem_t).wait()

    for i in range(_BPW // 16):
        sl = pl.ds(i * 16, 16)
        mvals = plsc.load_gather(ms_tab_v, [mid_v[sl]])
        g_v[sl] = ts_v[sl] + mvals
    pltpu.sync_copy(g_v, g_hbm.at[pl.ds(base, _BPW)])


_SC_GATHER = None


def _get_sc_gather():
    global _SC_GATHER
    if _SC_GATHER is None:
        _SC_GATHER = pl.kernel(
            _sc_gather_body,
            out_type=jax.ShapeDtypeStruct((_B,), jnp.float32),
            mesh=plsc.VectorSubcoreMesh(
                core_axis_name="c", subcore_axis_name="s",
                num_cores=_NC, num_subcores=_NS),
            scratch_types=[
                pltpu.VMEM((_BPW,), jnp.int32),
                pltpu.VMEM((_BPW,), jnp.int32),
                pltpu.VMEM((_BPW,), jnp.float32),
                pltpu.VMEM((_NM,), jnp.float32),
                pltpu.VMEM((_BPW,), jnp.float32),
                pltpu.VMEM((_NT,), jnp.float32),
                pltpu.VMEM_SHARED((_NT,), jnp.float32),
                pltpu.SemaphoreType.DMA,
            ],
            compiler_params=pltpu.CompilerParams(
                use_tc_tiling_on_sc=False, needs_layout_passes=False),
        )
    return _SC_GATHER


_TBLK = 32768


def _tc_scores_body(tt_ref, mt_ref, w_ref, ts_ref, ms_ref):
    wt = w_ref[_NH:_NH + _ED, :]
    ts_ref[...] = jnp.sum(tt_ref[...] * wt, axis=0)

    @pl.when(pl.program_id(0) == 0)
    def _():
        wm = w_ref[_NH + _ED:, :]
        ms_ref[...] = jnp.sum(mt_ref[...] * wm, axis=0)


_tc_scores = pl.pallas_call(
    _tc_scores_body,
    grid=(pl.cdiv(_NT, _TBLK),),
    in_specs=[
        pl.BlockSpec((_ED, _TBLK), lambda i: (0, i)),
        pl.BlockSpec((_ED, _NM), lambda i: (0, 0)),
        pl.BlockSpec((544, 1), lambda i: (0, 0)),
    ],
    out_specs=[
        pl.BlockSpec((_TBLK,), lambda i: (i,)),
        pl.BlockSpec((_NM,), lambda i: (0,)),
    ],
    out_shape=[
        jax.ShapeDtypeStruct((_NT,), jnp.float32),
        jax.ShapeDtypeStruct((_NM,), jnp.float32),
    ],
)

_BLK = 2048


def _tc_head_body(h_ref, w_ref, b_ref, o_ref):
    wh = w_ref[:_NH, :]
    acc = jnp.dot(h_ref[...], wh, preferred_element_type=jnp.float32)
    o_ref[...] = acc[:, 0] + b_ref[0]


_tc_head = pl.pallas_call(
    _tc_head_body,
    grid=(_B // _BLK,),
    in_specs=[
        pl.BlockSpec((_BLK, _NH), lambda i: (i, 0)),
        pl.BlockSpec((544, 1), lambda i: (0, 0)),
        pl.BlockSpec(memory_space=pltpu.SMEM),
    ],
    out_specs=pl.BlockSpec((_BLK,), lambda i: (i,)),
    out_shape=jax.ShapeDtypeStruct((_B,), jnp.float32),
)


@jax.jit
def kernel(h, teacher_id, materia_id, teacher_emb, materia_emb, W, b):
    tid = teacher_id.astype(jnp.int32)
    mid = materia_id.astype(jnp.int32)
    wT = W.T
    ts, ms = _tc_scores(teacher_emb.T, materia_emb.T, wT)
    g = _get_sc_gather()(ts, ms, tid, mid)
    oh = _tc_head(h, wT, b)
    return oh + g


# speedup vs baseline: 3.9523x; 1.0082x over previous
import jax
import jax.numpy as jnp
from jax import lax
from jax.experimental import pallas as pl
from jax.experimental.pallas import tpu as pltpu
from jax.experimental.pallas import tpu_sc as plsc

_B = 16384
_NH = 512
_ED = 16
_NT = 100000
_NM = 1000

_NC = 2
_NS = 16
_NW = _NC * _NS
_BPW = _B // _NW


def _sc_gather_body(ts_hbm, ms_hbm, tid_hbm, mid_hbm,
                    g_hbm,
                    tid_v, mid_v, ts_v, ms_tab_v, g_v, stage_v, ts_spm,
                    sem_t):
    sid = lax.axis_index("s")
    wid = sid * _NC + lax.axis_index("c")
    base = wid * _BPW
    pltpu.sync_copy(tid_hbm.at[pl.ds(base, _BPW)], tid_v)
    pltpu.sync_copy(mid_hbm.at[pl.ds(base, _BPW)], mid_v)
    pltpu.sync_copy(ms_hbm, ms_tab_v)

    @pl.when(sid == 0)
    def _():
        pltpu.sync_copy(ts_hbm, stage_v)
        pltpu.sync_copy(stage_v, ts_spm)

    plsc.subcore_barrier()
    pltpu.async_copy(ts_spm.at[tid_v], ts_v, sem_t).wait()

    for i in range(_BPW // 16):
        sl = pl.ds(i * 16, 16)
        mvals = plsc.load_gather(ms_tab_v, [mid_v[sl]])
        g_v[sl] = ts_v[sl] + mvals
    pltpu.sync_copy(g_v, g_hbm.at[pl.ds(base, _BPW)])


_SC_GATHER = None


def _get_sc_gather():
    global _SC_GATHER
    if _SC_GATHER is None:
        _SC_GATHER = pl.kernel(
            _sc_gather_body,
            out_type=jax.ShapeDtypeStruct((_B,), jnp.float32),
            mesh=plsc.VectorSubcoreMesh(
                core_axis_name="c", subcore_axis_name="s",
                num_cores=_NC, num_subcores=_NS),
            scratch_types=[
                pltpu.VMEM((_BPW,), jnp.int32),
                pltpu.VMEM((_BPW,), jnp.int32),
                pltpu.VMEM((_BPW,), jnp.float32),
                pltpu.VMEM((_NM,), jnp.float32),
                pltpu.VMEM((_BPW,), jnp.float32),
                pltpu.VMEM((_NT,), jnp.float32),
                pltpu.VMEM_SHARED((_NT,), jnp.float32),
                pltpu.SemaphoreType.DMA,
            ],
            compiler_params=pltpu.CompilerParams(
                use_tc_tiling_on_sc=False, needs_layout_passes=False),
        )
    return _SC_GATHER


_TBLK = 25600


def _tc_scores_body(tt_ref, mt_ref, w_ref, ts_ref, ms_ref):
    wt = w_ref[_NH:_NH + _ED, :]
    ts_ref[...] = jnp.sum(tt_ref[...] * wt, axis=0)

    @pl.when(pl.program_id(0) == 0)
    def _():
        wm = w_ref[_NH + _ED:, :]
        ms_ref[...] = jnp.sum(mt_ref[...] * wm, axis=0)


_tc_scores = pl.pallas_call(
    _tc_scores_body,
    grid=(pl.cdiv(_NT, _TBLK),),
    in_specs=[
        pl.BlockSpec((_ED, _TBLK), lambda i: (0, i)),
        pl.BlockSpec((_ED, _NM), lambda i: (0, 0)),
        pl.BlockSpec((544, 1), lambda i: (0, 0)),
    ],
    out_specs=[
        pl.BlockSpec((_TBLK,), lambda i: (i,)),
        pl.BlockSpec((_NM,), lambda i: (0,)),
    ],
    out_shape=[
        jax.ShapeDtypeStruct((_NT,), jnp.float32),
        jax.ShapeDtypeStruct((_NM,), jnp.float32),
    ],
)

_BLK = 2048


def _tc_head_body(h_ref, w_ref, b_ref, o_ref):
    wh = w_ref[:_NH, :]
    acc = jnp.dot(h_ref[...], wh, preferred_element_type=jnp.float32)
    o_ref[...] = acc[:, 0] + b_ref[0]


_tc_head = pl.pallas_call(
    _tc_head_body,
    grid=(_B // _BLK,),
    in_specs=[
        pl.BlockSpec((_BLK, _NH), lambda i: (i, 0)),
        pl.BlockSpec((544, 1), lambda i: (0, 0)),
        pl.BlockSpec(memory_space=pltpu.SMEM),
    ],
    out_specs=pl.BlockSpec((_BLK,), lambda i: (i,)),
    out_shape=jax.ShapeDtypeStruct((_B,), jnp.float32),
)


@jax.jit
def kernel(h, teacher_id, materia_id, teacher_emb, materia_emb, W, b):
    tid = teacher_id.astype(jnp.int32)
    mid = materia_id.astype(jnp.int32)
    wT = W.T
    ts, ms = _tc_scores(teacher_emb.T, materia_emb.T, wT)
    g = _get_sc_gather()(ts, ms, tid, mid)
    oh = _tc_head(h, wT, b)
    return oh + g


# speedup vs baseline: 4.0308x; 1.0198x over previous
import jax
import jax.numpy as jnp
from jax import lax
from jax.experimental import pallas as pl
from jax.experimental.pallas import tpu as pltpu
from jax.experimental.pallas import tpu_sc as plsc

_B = 16384
_NH = 512
_ED = 16
_NT = 100000
_NM = 1000

_NC = 2
_NS = 16
_NW = _NC * _NS
_BPW = _B // _NW


def _sc_gather_body(ts_hbm, ms_hbm, tid_hbm, mid_hbm,
                    g_hbm,
                    tid_v, mid_v, ts_v, ms_tab_v, g_v, stage_v, ts_spm,
                    sem_t):
    sid = lax.axis_index("s")
    wid = sid * _NC + lax.axis_index("c")
    base = wid * _BPW
    pltpu.sync_copy(tid_hbm.at[pl.ds(base, _BPW)], tid_v)
    pltpu.sync_copy(mid_hbm.at[pl.ds(base, _BPW)], mid_v)
    pltpu.sync_copy(ms_hbm, ms_tab_v)

    @pl.when(sid == 0)
    def _():
        pltpu.sync_copy(ts_hbm, stage_v)
        pltpu.sync_copy(stage_v, ts_spm)

    plsc.subcore_barrier()
    pltpu.async_copy(ts_spm.at[tid_v], ts_v, sem_t).wait()

    for i in range(_BPW // 16):
        sl = pl.ds(i * 16, 16)
        mvals = plsc.load_gather(ms_tab_v, [mid_v[sl]])
        g_v[sl] = ts_v[sl] + mvals
    pltpu.sync_copy(g_v, g_hbm.at[pl.ds(base, _BPW)])


_SC_GATHER = None


def _get_sc_gather():
    global _SC_GATHER
    if _SC_GATHER is None:
        _SC_GATHER = pl.kernel(
            _sc_gather_body,
            out_type=jax.ShapeDtypeStruct((_B,), jnp.float32),
            mesh=plsc.VectorSubcoreMesh(
                core_axis_name="c", subcore_axis_name="s",
                num_cores=_NC, num_subcores=_NS),
            scratch_types=[
                pltpu.VMEM((_BPW,), jnp.int32),
                pltpu.VMEM((_BPW,), jnp.int32),
                pltpu.VMEM((_BPW,), jnp.float32),
                pltpu.VMEM((_NM,), jnp.float32),
                pltpu.VMEM((_BPW,), jnp.float32),
                pltpu.VMEM((_NT,), jnp.float32),
                pltpu.VMEM_SHARED((_NT,), jnp.float32),
                pltpu.SemaphoreType.DMA,
            ],
            compiler_params=pltpu.CompilerParams(
                use_tc_tiling_on_sc=False, needs_layout_passes=False),
        )
    return _SC_GATHER


_TBLK = 25600


def _tc_scores_body(tt_ref, mt_ref, w_ref, ts_ref, ms_ref):
    wt = w_ref[_NH:_NH + _ED, :]
    ts_ref[...] = jnp.sum(tt_ref[...] * wt, axis=0)

    @pl.when(pl.program_id(0) == 0)
    def _():
        wm = w_ref[_NH + _ED:, :]
        ms_ref[...] = jnp.sum(mt_ref[...] * wm, axis=0)


_tc_scores = pl.pallas_call(
    _tc_scores_body,
    grid=(pl.cdiv(_NT, _TBLK),),
    in_specs=[
        pl.BlockSpec((_ED, _TBLK), lambda i: (0, i)),
        pl.BlockSpec((_ED, _NM), lambda i: (0, 0)),
        pl.BlockSpec((544, 1), lambda i: (0, 0)),
    ],
    out_specs=[
        pl.BlockSpec((_TBLK,), lambda i: (i,)),
        pl.BlockSpec((_NM,), lambda i: (0,)),
    ],
    out_shape=[
        jax.ShapeDtypeStruct((_NT,), jnp.float32),
        jax.ShapeDtypeStruct((_NM,), jnp.float32),
    ],
)

_BLK = 4096


def _tc_head_body(h_ref, w_ref, b_ref, o_ref):
    wh = w_ref[:_NH, :]
    acc = jnp.dot(h_ref[...], wh, preferred_element_type=jnp.float32)
    o_ref[...] = acc[:, 0] + b_ref[0]


_tc_head = pl.pallas_call(
    _tc_head_body,
    grid=(_B // _BLK,),
    in_specs=[
        pl.BlockSpec((_BLK, _NH), lambda i: (i, 0)),
        pl.BlockSpec((544, 1), lambda i: (0, 0)),
        pl.BlockSpec(memory_space=pltpu.SMEM),
    ],
    out_specs=pl.BlockSpec((_BLK,), lambda i: (i,)),
    out_shape=jax.ShapeDtypeStruct((_B,), jnp.float32),
)


@jax.jit
def kernel(h, teacher_id, materia_id, teacher_emb, materia_emb, W, b):
    tid = teacher_id.astype(jnp.int32)
    mid = materia_id.astype(jnp.int32)
    wT = W.T
    ts, ms = _tc_scores(teacher_emb.T, materia_emb.T, wT)
    g = _get_sc_gather()(ts, ms, tid, mid)
    oh = _tc_head(h, wT, b)
    return oh + g
